# baseline (speedup 1.0000x reference)
import jax
import jax.numpy as jnp
from jax.experimental import pallas as pl
from jax.experimental.pallas import tpu as pltpu

_BM = 400
_NBUF = 2
_NSPLIT = 2


def _gcn_body(a_hbm, e_ref, o_ref, a_buf, sems):
    nblk = a_hbm.shape[0] // _BM
    rows = _BM // _NSPLIT

    def copy(slot, idx, s):
        return pltpu.make_async_copy(
            a_hbm.at[pl.ds(idx * _BM + s * rows, rows), :],
            a_buf.at[slot, pl.ds(s * rows, rows), :],
            sems.at[slot, s],
        )

    def start(idx):
        for s in range(_NSPLIT):
            copy(idx % _NBUF, idx, s).start()

    def wait(idx):
        for s in range(_NSPLIT):
            copy(idx % _NBUF, idx, s).wait()

    for i in range(_NBUF - 1):
        start(i)
    for i in range(nblk):
        if i + _NBUF - 1 < nblk:
            start(i + _NBUF - 1)
        wait(i)
        o_ref[pl.ds(i * _BM, _BM), :] = jnp.dot(
            a_buf[i % _NBUF], e_ref[...], preferred_element_type=jnp.float32)


def kernel(A_hat, E):
    n, k = A_hat.shape
    d = E.shape[1]
    return pl.pallas_call(
        _gcn_body,
        in_specs=[
            pl.BlockSpec(memory_space=pltpu.MemorySpace.HBM),
            pl.BlockSpec(memory_space=pltpu.MemorySpace.VMEM),
        ],
        out_specs=pl.BlockSpec(memory_space=pltpu.MemorySpace.VMEM),
        out_shape=jax.ShapeDtypeStruct((n, d), jnp.float32),
        scratch_shapes=[
            pltpu.MemorySpace.VMEM((_NBUF, _BM, k), jnp.float32),
            pltpu.SemaphoreType.DMA((_NBUF, _NSPLIT)),
        ],
    )(A_hat, E)


# speedup vs baseline: 1.0700x; 1.0700x over previous
import jax
import jax.numpy as jnp
from jax.experimental import pallas as pl
from jax.experimental.pallas import tpu as pltpu

_BM = 80
_NBUF = 8


def _gcn_body(a_hbm, e_ref, o_ref, a_buf, sems):
    nblk = a_hbm.shape[0] // _BM

    def copy(slot, idx):
        return pltpu.make_async_copy(
            a_hbm.at[pl.ds(idx * _BM, _BM), :],
            a_buf.at[slot],
            sems.at[slot],
        )

    for i in range(_NBUF - 1):
        copy(i, i).start()

    def loop(i, carry):
        slot = jax.lax.rem(i, _NBUF)

        @pl.when(i + _NBUF - 1 < nblk)
        def _():
            copy(jax.lax.rem(i + _NBUF - 1, _NBUF), i + _NBUF - 1).start()

        copy(slot, i).wait()
        o_ref[pl.ds(i * _BM, _BM), :] = jnp.dot(
            a_buf[slot], e_ref[...], preferred_element_type=jnp.float32)
        return carry

    jax.lax.fori_loop(0, nblk, loop, 0)


def kernel(A_hat, E):
    n, k = A_hat.shape
    d = E.shape[1]
    return pl.pallas_call(
        _gcn_body,
        in_specs=[
            pl.BlockSpec(memory_space=pltpu.MemorySpace.HBM),
            pl.BlockSpec(memory_space=pltpu.MemorySpace.VMEM),
        ],
        out_specs=pl.BlockSpec(memory_space=pltpu.MemorySpace.VMEM),
        out_shape=jax.ShapeDtypeStruct((n, d), jnp.float32),
        scratch_shapes=[
            pltpu.MemorySpace.VMEM((_NBUF, _BM, k), jnp.float32),
            pltpu.SemaphoreType.DMA((_NBUF,)),
        ],
    )(A_hat, E)
